# baseline (speedup 1.0000x reference)
import functools

import jax
import jax.numpy as jnp
from jax import lax
from jax.experimental import pallas as pl
from jax.experimental.pallas import tpu as pltpu
from jax.experimental.pallas import tpu_sc as plsc

_N = 16384
_NCM1 = 9
_NC = 10
_BLKL = 8192

_NSC = 2
_NSUB = 16
_NW = _NSC * _NSUB
_CHUNK = _N // _NW


def _sc_hist_body(labels_hbm, out_hbm, lab_v, cnt_m):
    s = lax.axis_index("s") * _NSC + lax.axis_index("c")
    pltpu.sync_copy(labels_hbm.at[pl.ds(s * _CHUNK, _CHUNK)], lab_v)
    def _step(i, accs):
        lv = lab_v[pl.ds(i * 16, 16)]
        return tuple(
            accs[c] + jnp.where(lv == c, 1.0, 0.0) for c in range(_NC)
        )

    accs = lax.fori_loop(
        0,
        _CHUNK // 16,
        _step,
        tuple(jnp.zeros((16,), jnp.float32) for _ in range(_NC)),
    )
    zero = jnp.zeros((16,), jnp.float32)
    for c in range(16):
        cnt_m[c, :] = accs[c] if c < _NC else zero
    pltpu.sync_copy(cnt_m, out_hbm.at[pl.ds(s * 16, 16)])


@functools.cache
def _sc_hist():
    return pl.kernel(
        _sc_hist_body,
        out_type=jax.ShapeDtypeStruct((_NW * 16, 16), jnp.float32),
        mesh=plsc.VectorSubcoreMesh(
            core_axis_name="c",
            subcore_axis_name="s",
            num_cores=_NSC,
            num_subcores=_NSUB,
        ),
        scratch_types=[
            pltpu.VMEM((_CHUNK,), jnp.int32),
            pltpu.VMEM((16, 16), jnp.float32),
        ],
    )


def _sc_hist_call(labels_i32):
    return _sc_hist()(labels_i32)


def _tc_body(logits_ref, labels_ref, out_ref, acc_ref):
    step = pl.program_id(0)
    nsteps = pl.num_programs(0)

    @pl.when(step == 0)
    def _init():
        acc_ref[...] = jnp.zeros_like(acc_ref)

    sig = jax.nn.sigmoid(logits_ref[...])
    lab = labels_ref[...]
    row9 = lax.broadcasted_iota(jnp.int32, (_NCM1, _BLKL), 0)
    diffmask = (row9 == lab).astype(jnp.float32) - (row9 == lab + 1).astype(
        jnp.float32
    )
    gathered = jnp.sum(sig * diffmask, axis=0, keepdims=True) - (
        lab == _NCM1 - 1
    ).astype(jnp.float32)
    logt = jnp.log(gathered + 1e-9)

    row16 = lax.broadcasted_iota(jnp.int32, (16, _BLKL), 0)
    acc_ref[...] += (row16 == lab).astype(jnp.float32) * logt

    @pl.when(step == nsteps - 1)
    def _finalize():
        out_ref[...] = jnp.sum(acc_ref[...], axis=1, keepdims=True)


def _comb_body(counts_ref, ssum_ref, out_ref):
    ssum = ssum_ref[...]
    cnt_m = counts_ref[pl.ds(0, 16), :]
    for t in range(1, _NW):
        cnt_m += counts_ref[pl.ds(t * 16, 16), :]
    counts = jnp.sum(cnt_m, axis=1, keepdims=True)
    cls = lax.broadcasted_iota(jnp.int32, (16, 1), 0)
    valid = cls < _NC
    total = jnp.sum(jnp.where(valid, counts, 0.0))
    w = counts / total
    w = jnp.where(w == 0.0, 1.0, w)
    inv = jnp.where(valid, 1.0 / w, 0.0)
    inv = inv / jnp.sum(inv)
    out_ref[...] = jnp.reshape(-jnp.sum(inv * ssum) / _N, (1, 1))


@jax.jit
def _loss(logits_t, labels2d, counts):
    grid = _N // _BLKL
    ssum = pl.pallas_call(
        _tc_body,
        grid=(grid,),
        in_specs=[
            pl.BlockSpec((_NCM1, _BLKL), lambda i: (0, i)),
            pl.BlockSpec((1, _BLKL), lambda i: (0, i)),
        ],
        out_specs=pl.BlockSpec((16, 1), lambda i: (0, 0)),
        out_shape=jax.ShapeDtypeStruct((16, 1), jnp.float32),
        scratch_shapes=[pltpu.VMEM((16, _BLKL), jnp.float32)],
    )(logits_t, labels2d)
    out = pl.pallas_call(
        _comb_body,
        out_shape=jax.ShapeDtypeStruct((1, 1), jnp.float32),
    )(counts, ssum)
    return out[0, 0]


def kernel(logits, labels):
    logits_t = logits.reshape(-1, _NCM1).T
    labels_i32 = labels.reshape(-1).astype(jnp.int32)
    labels2d = labels_i32.reshape(1, -1)
    counts = _sc_hist_call(labels_i32)
    return _loss(logits_t, labels2d, counts)


# speedup vs baseline: 1.0598x; 1.0598x over previous
import functools

import jax
import jax.numpy as jnp
from jax import lax
from jax.experimental import pallas as pl
from jax.experimental.pallas import tpu as pltpu
from jax.experimental.pallas import tpu_sc as plsc

_N = 16384
_NCM1 = 9
_NC = 10
_BLKL = 8192

_NW = 16
_CHUNK = _N // _NW


def _sc_hist_body(labels_hbm, out_hbm, lab_v, cnt_m):
    s = lax.axis_index("s")
    pltpu.sync_copy(labels_hbm.at[pl.ds(s * _CHUNK, _CHUNK)], lab_v)
    def _step(i, accs):
        lv = lab_v[pl.ds(i * 16, 16)]
        return tuple(
            accs[c] + jnp.where(lv == c, 1.0, 0.0) for c in range(_NC)
        )

    accs = lax.fori_loop(
        0,
        _CHUNK // 16,
        _step,
        tuple(jnp.zeros((16,), jnp.float32) for _ in range(_NC)),
    )
    zero = jnp.zeros((16,), jnp.float32)
    for c in range(16):
        cnt_m[c, :] = accs[c] if c < _NC else zero
    pltpu.sync_copy(cnt_m, out_hbm.at[pl.ds(s * 16, 16)])


@functools.cache
def _sc_hist():
    return pl.kernel(
        _sc_hist_body,
        out_type=jax.ShapeDtypeStruct((_NW * 16, 16), jnp.float32),
        mesh=plsc.VectorSubcoreMesh(
            core_axis_name="c",
            subcore_axis_name="s",
            num_cores=1,
            num_subcores=_NW,
        ),
        scratch_types=[
            pltpu.VMEM((_CHUNK,), jnp.int32),
            pltpu.VMEM((16, 16), jnp.float32),
        ],
    )


def _sc_hist_call(labels_i32):
    return _sc_hist()(labels_i32)


def _tc_body(logits_ref, labels_ref, out_ref, acc_ref):
    step = pl.program_id(0)
    nsteps = pl.num_programs(0)

    @pl.when(step == 0)
    def _init():
        acc_ref[...] = jnp.zeros_like(acc_ref)

    sig = jax.nn.sigmoid(logits_ref[...])
    lab = labels_ref[...]
    row9 = lax.broadcasted_iota(jnp.int32, (_NCM1, _BLKL), 0)
    diffmask = (row9 == lab).astype(jnp.float32) - (row9 == lab + 1).astype(
        jnp.float32
    )
    gathered = jnp.sum(sig * diffmask, axis=0, keepdims=True) - (
        lab == _NCM1 - 1
    ).astype(jnp.float32)
    logt = jnp.log(gathered + 1e-9)

    row16 = lax.broadcasted_iota(jnp.int32, (16, _BLKL), 0)
    acc_ref[...] += (row16 == lab).astype(jnp.float32) * logt

    @pl.when(step == nsteps - 1)
    def _finalize():
        out_ref[...] = jnp.sum(acc_ref[...], axis=1, keepdims=True)


def _comb_body(counts_ref, ssum_ref, out_ref):
    ssum = ssum_ref[...]
    cnt_m = counts_ref[pl.ds(0, 16), :]
    for t in range(1, _NW):
        cnt_m += counts_ref[pl.ds(t * 16, 16), :]
    counts = jnp.sum(cnt_m, axis=1, keepdims=True)
    cls = lax.broadcasted_iota(jnp.int32, (16, 1), 0)
    valid = cls < _NC
    total = jnp.sum(jnp.where(valid, counts, 0.0))
    w = counts / total
    w = jnp.where(w == 0.0, 1.0, w)
    inv = jnp.where(valid, 1.0 / w, 0.0)
    inv = inv / jnp.sum(inv)
    out_ref[...] = jnp.reshape(-jnp.sum(inv * ssum) / _N, (1, 1))


@jax.jit
def _loss(logits_t, labels2d, counts):
    grid = _N // _BLKL
    ssum = pl.pallas_call(
        _tc_body,
        grid=(grid,),
        in_specs=[
            pl.BlockSpec((_NCM1, _BLKL), lambda i: (0, i)),
            pl.BlockSpec((1, _BLKL), lambda i: (0, i)),
        ],
        out_specs=pl.BlockSpec((16, 1), lambda i: (0, 0)),
        out_shape=jax.ShapeDtypeStruct((16, 1), jnp.float32),
        scratch_shapes=[pltpu.VMEM((16, _BLKL), jnp.float32)],
    )(logits_t, labels2d)
    out = pl.pallas_call(
        _comb_body,
        out_shape=jax.ShapeDtypeStruct((1, 1), jnp.float32),
    )(counts, ssum)
    return out[0, 0]


def kernel(logits, labels):
    logits_t = logits.reshape(-1, _NCM1).T
    labels_i32 = labels.reshape(-1).astype(jnp.int32)
    labels2d = labels_i32.reshape(1, -1)
    counts = _sc_hist_call(labels_i32)
    return _loss(logits_t, labels2d, counts)
